# baseline (speedup 1.0000x reference)
import functools
import math

import jax
import jax.numpy as jnp
from jax import lax
from jax.experimental import pallas as pl
from jax.experimental.pallas import tpu as pltpu
from jax.experimental.pallas import tpu_sc as plsc

_SIZE = 32000
_PAD_IDX = 0
_SMOOTHING = 0.1
_CONFIDENCE = 1.0 - _SMOOTHING
_S = _SMOOTHING / (_SIZE - 2)
_C0 = _SMOOTHING * math.log(_S) + _CONFIDENCE * math.log(_CONFIDENCE)

_N_TOK = 8192
_BR = 64
_NT = 6400
_NC, _NS = 2, 16
_NW = _NC * _NS
_KW = (_N_TOK - _NT) // _NW
_NG = _KW // 8
_CB = 3200
_NCHUNK = _SIZE // _CB


def _tc_kernel(t_ref, x_ref, o_ref):
    ri = pl.program_id(0)

    @pl.when(ri == 0)
    def _init():
        o_ref[...] = jnp.zeros_like(o_ref)

    x = x_ref[...]
    t = t_ref[...]
    maskf = (t != _PAD_IDX).astype(jnp.float32)
    cols = lax.broadcasted_iota(jnp.int32, x.shape, 1)
    coef = jnp.where(cols == t, jnp.float32(-_CONFIDENCE), jnp.float32(-_S))
    wrow = jnp.sum(x * coef, axis=1, keepdims=True)
    part = jnp.sum(maskf * (wrow + jnp.float32(_S) * x[:, 0:1] + jnp.float32(_C0)))
    o_ref[...] += part.reshape(1, 1)


def _sc_chunk_accum(buf_v, tb, carry, ccol0):
    rs = list(carry[0:8])
    tv = list(carry[8:16])
    viota = lax.iota(jnp.int32, 16)

    def jbody(j, c):
        rs_l = list(c[0:8])
        tv_l = list(c[8:16])
        for u in range(8):
            off = j * 128 + u * 16
            colv = ccol0 + off + viota
            for i in range(8):
                xv = buf_v[i, pl.ds(off, 16)]
                rs_l[i] = rs_l[i] + xv
                m = colv == tb[i]
                tv_l[i] = tv_l[i] + jnp.where(m, xv, jnp.float32(0.0))
        return tuple(rs_l) + tuple(tv_l)

    out = lax.fori_loop(0, _CB // 128, jbody, tuple(rs) + tuple(tv))
    return out


def _sc_body(x_hbm, x0_hbm, t_hbm, o_hbm,
             bufa, bufb, t_stage, x0_stage, acc_v, sema, semb):
    c = lax.axis_index("c")
    s = lax.axis_index("s")
    wid = s * _NC + c
    base_row = _NT + wid * _KW
    pltpu.sync_copy(t_hbm.at[pl.ds(base_row, _KW)], t_stage.at[pl.ds(0, _KW)])
    pltpu.sync_copy(x0_hbm.at[pl.ds(base_row, _KW)], x0_stage.at[pl.ds(0, _KW)])
    viota = lax.iota(jnp.int32, 16)

    def group(g, wacc):
        row0 = base_row + g * 8
        tvec = t_stage[pl.ds(g * 8, 16)]
        x0vec = x0_stage[pl.ds(g * 8, 16)]
        dnums = lax.GatherDimensionNumbers(
            offset_dims=(), collapsed_slice_dims=(0,), start_index_map=(0,))
        tb = [lax.gather(tvec, jnp.full((16, 1), i, jnp.int32), dnums,
                         slice_sizes=(1,),
                         mode=lax.GatherScatterMode.PROMISE_IN_BOUNDS)
              for i in range(8)]

        zero = jnp.zeros((16,), jnp.float32)
        carry0 = tuple(zero for _ in range(16))

        pltpu.async_copy(
            x_hbm.at[pl.ds(row0, 8), pl.ds(0, _CB)], bufa, sema)

        def pair(k, carry):
            cc0 = 2 * k
            cc1 = 2 * k + 1
            pltpu.async_copy(
                x_hbm.at[pl.ds(row0, 8), pl.ds(cc1 * _CB, _CB)], bufb, semb)
            pltpu.make_async_copy(
                x_hbm.at[pl.ds(row0, 8), pl.ds(cc0 * _CB, _CB)], bufa,
                sema).wait()
            carry = _sc_chunk_accum(bufa, tb, carry, cc0 * _CB)

            @pl.when(k < _NCHUNK // 2 - 1)
            def _():
                pltpu.async_copy(
                    x_hbm.at[pl.ds(row0, 8), pl.ds((cc0 + 2) * _CB, _CB)],
                    bufa, sema)

            pltpu.make_async_copy(
                x_hbm.at[pl.ds(row0, 8), pl.ds(cc1 * _CB, _CB)], bufb,
                semb).wait()
            carry = _sc_chunk_accum(bufb, tb, carry, cc1 * _CB)
            return carry

        carry = lax.fori_loop(0, _NCHUNK // 2, pair, carry0)

        rsv = jnp.zeros((16,), jnp.float32)
        tvv = jnp.zeros((16,), jnp.float32)
        for i in range(8):
            maskf = jnp.where(tb[i] != _PAD_IDX, jnp.float32(1.0),
                              jnp.float32(0.0))
            rsv = rsv + carry[i] * maskf
            tvv = tvv + carry[8 + i] * maskf
        gvec = jnp.float32(-_S) * rsv + jnp.float32(_S - _CONFIDENCE) * tvv
        percol = jnp.where(tvec != _PAD_IDX,
                           jnp.float32(_C0) + jnp.float32(_S) * x0vec,
                           jnp.float32(0.0))
        gvec = gvec + jnp.where(viota < 8, percol, jnp.float32(0.0))
        return wacc + gvec

    wtotal = lax.fori_loop(0, _NG, group, jnp.zeros((16,), jnp.float32))
    acc_v[...] = wtotal
    pltpu.sync_copy(acc_v, o_hbm.at[wid])


@functools.partial(
    pl.kernel,
    mesh=plsc.VectorSubcoreMesh(core_axis_name="c", subcore_axis_name="s"),
    out_type=jax.ShapeDtypeStruct((_NW, 16), jnp.float32),
    scratch_types=[
        pltpu.VMEM((8, _CB), jnp.float32),
        pltpu.VMEM((8, _CB), jnp.float32),
        pltpu.VMEM((_KW + 16,), jnp.int32),
        pltpu.VMEM((_KW + 16,), jnp.float32),
        pltpu.VMEM((16,), jnp.float32),
        pltpu.SemaphoreType.DMA,
        pltpu.SemaphoreType.DMA,
    ],
)
def _sc_stream(x_hbm, x0_hbm, t_hbm, o_hbm,
               bufa, bufb, t_stage, x0_stage, acc_v, sema, semb):
    _sc_body(x_hbm, x0_hbm, t_hbm, o_hbm,
             bufa, bufb, t_stage, x0_stage, acc_v, sema, semb)


def _combine_kernel(a_ref, b_ref, o_ref):
    o_ref[...] = a_ref[...] + jnp.sum(b_ref[...]).reshape(1, 1)


def kernel(x, target):
    n, v = x.shape
    t2 = target.reshape(n, 1)
    tc_part = pl.pallas_call(
        _tc_kernel,
        grid=(_NT // _BR,),
        in_specs=[
            pl.BlockSpec((_BR, 1), lambda i: (i, 0)),
            pl.BlockSpec((_BR, v), lambda i: (i, 0)),
        ],
        out_specs=pl.BlockSpec((1, 1), lambda i: (0, 0)),
        out_shape=jax.ShapeDtypeStruct((1, 1), jnp.float32),
    )(t2, x)
    x0col = lax.slice(x, (0, 0), (n, 1)).reshape(-1)
    sc_parts = _sc_stream(x, x0col, target)
    out = pl.pallas_call(
        _combine_kernel,
        out_shape=jax.ShapeDtypeStruct((1, 1), jnp.float32),
    )(tc_part, sc_parts)
    return out.reshape(())


# speedup vs baseline: 1.0006x; 1.0006x over previous
import functools
import math

import jax
import jax.numpy as jnp
from jax import lax
from jax.experimental import pallas as pl
from jax.experimental.pallas import tpu as pltpu
from jax.experimental.pallas import tpu_sc as plsc

_SIZE = 32000
_PAD_IDX = 0
_SMOOTHING = 0.1
_CONFIDENCE = 1.0 - _SMOOTHING
_S = _SMOOTHING / (_SIZE - 2)
_C0 = _SMOOTHING * math.log(_S) + _CONFIDENCE * math.log(_CONFIDENCE)

_N_TOK = 8192
_BR = 64
_NT = 7168
_NC, _NS = 2, 16
_NW = _NC * _NS
_KW = (_N_TOK - _NT) // _NW
_NG = _KW // 8
_CB = 3200
_NCHUNK = _SIZE // _CB


def _tc_kernel(t_ref, x_ref, o_ref):
    ri = pl.program_id(0)

    @pl.when(ri == 0)
    def _init():
        o_ref[...] = jnp.zeros_like(o_ref)

    x = x_ref[...]
    t = t_ref[...]
    maskf = (t != _PAD_IDX).astype(jnp.float32)
    cols = lax.broadcasted_iota(jnp.int32, x.shape, 1)
    coef = jnp.where(cols == t, jnp.float32(-_CONFIDENCE), jnp.float32(-_S))
    wrow = jnp.sum(x * coef, axis=1, keepdims=True)
    part = jnp.sum(maskf * (wrow + jnp.float32(_S) * x[:, 0:1] + jnp.float32(_C0)))
    o_ref[...] += part.reshape(1, 1)


def _sc_chunk_accum(buf_v, tb, carry, ccol0):
    rs = list(carry[0:8])
    tv = list(carry[8:16])
    viota = lax.iota(jnp.int32, 16)

    def jbody(j, c):
        rs_l = list(c[0:8])
        tv_l = list(c[8:16])
        for u in range(8):
            off = j * 128 + u * 16
            colv = ccol0 + off + viota
            for i in range(8):
                xv = buf_v[i, pl.ds(off, 16)]
                rs_l[i] = rs_l[i] + xv
                m = colv == tb[i]
                tv_l[i] = tv_l[i] + jnp.where(m, xv, jnp.float32(0.0))
        return tuple(rs_l) + tuple(tv_l)

    out = lax.fori_loop(0, _CB // 128, jbody, tuple(rs) + tuple(tv))
    return out


def _sc_body(x_hbm, x0_hbm, t_hbm, o_hbm,
             bufa, bufb, t_stage, x0_stage, acc_v, sema, semb):
    c = lax.axis_index("c")
    s = lax.axis_index("s")
    wid = s * _NC + c
    base_row = _NT + wid * _KW
    pltpu.sync_copy(t_hbm.at[pl.ds(base_row, _KW)], t_stage.at[pl.ds(0, _KW)])
    pltpu.sync_copy(x0_hbm.at[pl.ds(base_row, _KW)], x0_stage.at[pl.ds(0, _KW)])
    viota = lax.iota(jnp.int32, 16)

    def group(g, wacc):
        row0 = base_row + g * 8
        tvec = t_stage[pl.ds(g * 8, 16)]
        x0vec = x0_stage[pl.ds(g * 8, 16)]
        dnums = lax.GatherDimensionNumbers(
            offset_dims=(), collapsed_slice_dims=(0,), start_index_map=(0,))
        tb = [lax.gather(tvec, jnp.full((16, 1), i, jnp.int32), dnums,
                         slice_sizes=(1,),
                         mode=lax.GatherScatterMode.PROMISE_IN_BOUNDS)
              for i in range(8)]

        zero = jnp.zeros((16,), jnp.float32)
        carry0 = tuple(zero for _ in range(16))

        pltpu.async_copy(
            x_hbm.at[pl.ds(row0, 8), pl.ds(0, _CB)], bufa, sema)

        def pair(k, carry):
            cc0 = 2 * k
            cc1 = 2 * k + 1
            pltpu.async_copy(
                x_hbm.at[pl.ds(row0, 8), pl.ds(cc1 * _CB, _CB)], bufb, semb)
            pltpu.make_async_copy(
                x_hbm.at[pl.ds(row0, 8), pl.ds(cc0 * _CB, _CB)], bufa,
                sema).wait()
            carry = _sc_chunk_accum(bufa, tb, carry, cc0 * _CB)

            @pl.when(k < _NCHUNK // 2 - 1)
            def _():
                pltpu.async_copy(
                    x_hbm.at[pl.ds(row0, 8), pl.ds((cc0 + 2) * _CB, _CB)],
                    bufa, sema)

            pltpu.make_async_copy(
                x_hbm.at[pl.ds(row0, 8), pl.ds(cc1 * _CB, _CB)], bufb,
                semb).wait()
            carry = _sc_chunk_accum(bufb, tb, carry, cc1 * _CB)
            return carry

        carry = lax.fori_loop(0, _NCHUNK // 2, pair, carry0)

        rsv = jnp.zeros((16,), jnp.float32)
        tvv = jnp.zeros((16,), jnp.float32)
        for i in range(8):
            maskf = jnp.where(tb[i] != _PAD_IDX, jnp.float32(1.0),
                              jnp.float32(0.0))
            rsv = rsv + carry[i] * maskf
            tvv = tvv + carry[8 + i] * maskf
        gvec = jnp.float32(-_S) * rsv + jnp.float32(_S - _CONFIDENCE) * tvv
        percol = jnp.where(tvec != _PAD_IDX,
                           jnp.float32(_C0) + jnp.float32(_S) * x0vec,
                           jnp.float32(0.0))
        gvec = gvec + jnp.where(viota < 8, percol, jnp.float32(0.0))
        return wacc + gvec

    wtotal = lax.fori_loop(0, _NG, group, jnp.zeros((16,), jnp.float32))
    acc_v[...] = wtotal
    pltpu.sync_copy(acc_v, o_hbm.at[wid])


@functools.partial(
    pl.kernel,
    mesh=plsc.VectorSubcoreMesh(core_axis_name="c", subcore_axis_name="s"),
    out_type=jax.ShapeDtypeStruct((_NW, 16), jnp.float32),
    scratch_types=[
        pltpu.VMEM((8, _CB), jnp.float32),
        pltpu.VMEM((8, _CB), jnp.float32),
        pltpu.VMEM((_KW + 16,), jnp.int32),
        pltpu.VMEM((_KW + 16,), jnp.float32),
        pltpu.VMEM((16,), jnp.float32),
        pltpu.SemaphoreType.DMA,
        pltpu.SemaphoreType.DMA,
    ],
)
def _sc_stream(x_hbm, x0_hbm, t_hbm, o_hbm,
               bufa, bufb, t_stage, x0_stage, acc_v, sema, semb):
    _sc_body(x_hbm, x0_hbm, t_hbm, o_hbm,
             bufa, bufb, t_stage, x0_stage, acc_v, sema, semb)


def _combine_kernel(a_ref, b_ref, o_ref):
    o_ref[...] = a_ref[...] + jnp.sum(b_ref[...]).reshape(1, 1)


def kernel(x, target):
    n, v = x.shape
    t2 = target.reshape(n, 1)
    tc_part = pl.pallas_call(
        _tc_kernel,
        grid=(_NT // _BR,),
        in_specs=[
            pl.BlockSpec((_BR, 1), lambda i: (i, 0)),
            pl.BlockSpec((_BR, v), lambda i: (i, 0)),
        ],
        out_specs=pl.BlockSpec((1, 1), lambda i: (0, 0)),
        out_shape=jax.ShapeDtypeStruct((1, 1), jnp.float32),
    )(t2, x)
    x0col = lax.slice(x, (0, 0), (n, 1)).reshape(-1)
    sc_parts = _sc_stream(x, x0col, target)
    out = pl.pallas_call(
        _combine_kernel,
        out_shape=jax.ShapeDtypeStruct((1, 1), jnp.float32),
    )(tc_part, sc_parts)
    return out.reshape(())


# speedup vs baseline: 1.0016x; 1.0010x over previous
import functools
import math

import jax
import jax.numpy as jnp
from jax import lax
from jax.experimental import pallas as pl
from jax.experimental.pallas import tpu as pltpu
from jax.experimental.pallas import tpu_sc as plsc

_SIZE = 32000
_PAD_IDX = 0
_SMOOTHING = 0.1
_CONFIDENCE = 1.0 - _SMOOTHING
_S = _SMOOTHING / (_SIZE - 2)
_C0 = _SMOOTHING * math.log(_S) + _CONFIDENCE * math.log(_CONFIDENCE)

_N_TOK = 8192
_BR = 64
_NT = 5632
_NC, _NS = 2, 16
_NW = _NC * _NS
_KW = (_N_TOK - _NT) // _NW
_NG = _KW // 8
_CB = 3200
_NCHUNK = _SIZE // _CB


def _tc_kernel(t_ref, x_ref, o_ref):
    ri = pl.program_id(0)

    @pl.when(ri == 0)
    def _init():
        o_ref[...] = jnp.zeros_like(o_ref)

    x = x_ref[...]
    t = t_ref[...]
    maskf = (t != _PAD_IDX).astype(jnp.float32)
    cols = lax.broadcasted_iota(jnp.int32, x.shape, 1)
    coef = jnp.where(cols == t, jnp.float32(-_CONFIDENCE), jnp.float32(-_S))
    wrow = jnp.sum(x * coef, axis=1, keepdims=True)
    part = jnp.sum(maskf * (wrow + jnp.float32(_S) * x[:, 0:1] + jnp.float32(_C0)))
    o_ref[...] += part.reshape(1, 1)


def _sc_chunk_accum(buf_v, tb, carry, ccol0):
    rs = list(carry[0:8])
    tv = list(carry[8:16])
    viota = lax.iota(jnp.int32, 16)

    def jbody(j, c):
        rs_l = list(c[0:8])
        tv_l = list(c[8:16])
        for u in range(8):
            off = j * 128 + u * 16
            colv = ccol0 + off + viota
            for i in range(8):
                xv = buf_v[i, pl.ds(off, 16)]
                rs_l[i] = rs_l[i] + xv
                m = colv == tb[i]
                tv_l[i] = tv_l[i] + jnp.where(m, xv, jnp.float32(0.0))
        return tuple(rs_l) + tuple(tv_l)

    out = lax.fori_loop(0, _CB // 128, jbody, tuple(rs) + tuple(tv))
    return out


def _sc_body(x_hbm, x0_hbm, t_hbm, o_hbm,
             bufa, bufb, t_stage, x0_stage, acc_v, sema, semb):
    c = lax.axis_index("c")
    s = lax.axis_index("s")
    wid = s * _NC + c
    base_row = _NT + wid * _KW
    pltpu.sync_copy(t_hbm.at[pl.ds(base_row, _KW)], t_stage.at[pl.ds(0, _KW)])
    pltpu.sync_copy(x0_hbm.at[pl.ds(base_row, _KW)], x0_stage.at[pl.ds(0, _KW)])
    viota = lax.iota(jnp.int32, 16)

    def group(g, wacc):
        row0 = base_row + g * 8
        tvec = t_stage[pl.ds(g * 8, 16)]
        x0vec = x0_stage[pl.ds(g * 8, 16)]
        dnums = lax.GatherDimensionNumbers(
            offset_dims=(), collapsed_slice_dims=(0,), start_index_map=(0,))
        tb = [lax.gather(tvec, jnp.full((16, 1), i, jnp.int32), dnums,
                         slice_sizes=(1,),
                         mode=lax.GatherScatterMode.PROMISE_IN_BOUNDS)
              for i in range(8)]

        zero = jnp.zeros((16,), jnp.float32)
        carry0 = tuple(zero for _ in range(16))

        pltpu.async_copy(
            x_hbm.at[pl.ds(row0, 8), pl.ds(0, _CB)], bufa, sema)

        def pair(k, carry):
            cc0 = 2 * k
            cc1 = 2 * k + 1
            pltpu.async_copy(
                x_hbm.at[pl.ds(row0, 8), pl.ds(cc1 * _CB, _CB)], bufb, semb)
            pltpu.make_async_copy(
                x_hbm.at[pl.ds(row0, 8), pl.ds(cc0 * _CB, _CB)], bufa,
                sema).wait()
            carry = _sc_chunk_accum(bufa, tb, carry, cc0 * _CB)

            @pl.when(k < _NCHUNK // 2 - 1)
            def _():
                pltpu.async_copy(
                    x_hbm.at[pl.ds(row0, 8), pl.ds((cc0 + 2) * _CB, _CB)],
                    bufa, sema)

            pltpu.make_async_copy(
                x_hbm.at[pl.ds(row0, 8), pl.ds(cc1 * _CB, _CB)], bufb,
                semb).wait()
            carry = _sc_chunk_accum(bufb, tb, carry, cc1 * _CB)
            return carry

        carry = lax.fori_loop(0, _NCHUNK // 2, pair, carry0)

        rsv = jnp.zeros((16,), jnp.float32)
        tvv = jnp.zeros((16,), jnp.float32)
        for i in range(8):
            maskf = jnp.where(tb[i] != _PAD_IDX, jnp.float32(1.0),
                              jnp.float32(0.0))
            rsv = rsv + carry[i] * maskf
            tvv = tvv + carry[8 + i] * maskf
        gvec = jnp.float32(-_S) * rsv + jnp.float32(_S - _CONFIDENCE) * tvv
        percol = jnp.where(tvec != _PAD_IDX,
                           jnp.float32(_C0) + jnp.float32(_S) * x0vec,
                           jnp.float32(0.0))
        gvec = gvec + jnp.where(viota < 8, percol, jnp.float32(0.0))
        return wacc + gvec

    wtotal = lax.fori_loop(0, _NG, group, jnp.zeros((16,), jnp.float32))
    acc_v[...] = wtotal
    pltpu.sync_copy(acc_v, o_hbm.at[wid])


@functools.partial(
    pl.kernel,
    mesh=plsc.VectorSubcoreMesh(core_axis_name="c", subcore_axis_name="s"),
    out_type=jax.ShapeDtypeStruct((_NW, 16), jnp.float32),
    scratch_types=[
        pltpu.VMEM((8, _CB), jnp.float32),
        pltpu.VMEM((8, _CB), jnp.float32),
        pltpu.VMEM((_KW + 16,), jnp.int32),
        pltpu.VMEM((_KW + 16,), jnp.float32),
        pltpu.VMEM((16,), jnp.float32),
        pltpu.SemaphoreType.DMA,
        pltpu.SemaphoreType.DMA,
    ],
)
def _sc_stream(x_hbm, x0_hbm, t_hbm, o_hbm,
               bufa, bufb, t_stage, x0_stage, acc_v, sema, semb):
    _sc_body(x_hbm, x0_hbm, t_hbm, o_hbm,
             bufa, bufb, t_stage, x0_stage, acc_v, sema, semb)


def _combine_kernel(a_ref, b_ref, o_ref):
    o_ref[...] = a_ref[...] + jnp.sum(b_ref[...]).reshape(1, 1)


def kernel(x, target):
    n, v = x.shape
    t2 = target.reshape(n, 1)
    tc_part = pl.pallas_call(
        _tc_kernel,
        grid=(_NT // _BR,),
        in_specs=[
            pl.BlockSpec((_BR, 1), lambda i: (i, 0)),
            pl.BlockSpec((_BR, v), lambda i: (i, 0)),
        ],
        out_specs=pl.BlockSpec((1, 1), lambda i: (0, 0)),
        out_shape=jax.ShapeDtypeStruct((1, 1), jnp.float32),
    )(t2, x)
    x0col = lax.slice(x, (0, 0), (n, 1)).reshape(-1)
    sc_parts = _sc_stream(x, x0col, target)
    out = pl.pallas_call(
        _combine_kernel,
        out_shape=jax.ShapeDtypeStruct((1, 1), jnp.float32),
    )(tc_part, sc_parts)
    return out.reshape(())
